# baseline (speedup 1.0000x reference)
import functools

import jax
import jax.numpy as jnp
from jax import lax
from jax.experimental import pallas as pl
from jax.experimental.pallas import tpu as pltpu
from jax.experimental.pallas import tpu_sc as plsc

_L = 200
_D = 64
_LANES = 16


def _emb_body(tok_hbm, idx_hbm, pos_hbm, out_hbm, pos_v, idx_v, rows_v, sem,
              *, seqs_per_w, num_cores):
    wid = lax.axis_index("s") * num_cores + lax.axis_index("c")
    base = wid * (seqs_per_w * _L)
    pltpu.sync_copy(pos_hbm, pos_v)

    def seq_body(s, carry):
        off = base + s * _L
        pltpu.sync_copy(idx_hbm.at[pl.ds(off, _L)], idx_v)
        pltpu.async_copy(tok_hbm.at[idx_v], rows_v, sem).wait()

        def add_row(l, c2):
            for c in range(_D // _LANES):
                sl = pl.ds(c * _LANES, _LANES)
                rows_v[l, sl] = rows_v[l, sl] + pos_v[l, sl]
            return c2

        lax.fori_loop(0, _L, add_row, 0, unroll=2)
        pltpu.sync_copy(rows_v, out_hbm.at[pl.ds(off, _L)])
        return carry

    lax.fori_loop(0, seqs_per_w, seq_body, 0)


def kernel(x, token_table, pos_table):
    B, L = x.shape
    V, D = token_table.shape
    assert L == _L and D == _D
    info = plsc.get_sparse_core_info()
    nw = info.num_cores * info.num_subcores
    assert B % nw == 0
    seqs_per_w = B // nw

    flat_idx = x.reshape(B * L).astype(jnp.int32)
    mesh = plsc.VectorSubcoreMesh(core_axis_name="c", subcore_axis_name="s")

    emb = functools.partial(
        pl.kernel,
        mesh=mesh,
        out_type=jax.ShapeDtypeStruct((B * L, D), jnp.float32),
        scratch_types=[
            pltpu.VMEM((_L, _D), jnp.float32),
            pltpu.VMEM((_L,), jnp.int32),
            pltpu.VMEM((_L, _D), jnp.float32),
            pltpu.SemaphoreType.DMA,
        ],
        compiler_params=pltpu.CompilerParams(use_tc_tiling_on_sc=False),
    )(functools.partial(_emb_body, seqs_per_w=seqs_per_w,
                        num_cores=info.num_cores))

    out = emb(token_table, flat_idx, pos_table)
    return out.reshape(B, L, D)


# speedup vs baseline: 1.1768x; 1.1768x over previous
import functools

import jax
import jax.numpy as jnp
from jax import lax
from jax.experimental import pallas as pl
from jax.experimental.pallas import tpu as pltpu
from jax.experimental.pallas import tpu_sc as plsc

_L = 200
_D = 64
_LANES = 16
_NBUF = 4
_DEPTH = 2
_G0, _G1 = 128, _L - 128


def _emb_body(tok_hbm, idx_hbm, pos_hbm, out_hbm,
              idx_all, pos_v, b0, b1, b2, b3,
              g0, g1, g2, g3, o0, o1, o2, o3,
              *, seqs_per_w, num_cores):
    bufs = (b0, b1, b2, b3)
    gsems = (g0, g1, g2, g3)
    osems = (o0, o1, o2, o3)
    nrows = seqs_per_w * _L
    wid = lax.axis_index("s") * num_cores + lax.axis_index("c")
    base = pl.multiple_of(wid * nrows, _L)

    pltpu.sync_copy(pos_hbm, pos_v)
    pltpu.sync_copy(idx_hbm.at[pl.ds(base, nrows)], idx_all)

    def start_gather(c, b):
        off = pl.multiple_of(c * _L, 8)
        pltpu.async_copy(tok_hbm.at[idx_all.at[pl.ds(off, _G0)]],
                         bufs[b].at[pl.ds(0, _G0)], gsems[b])
        off1 = pl.multiple_of(c * _L + _G0, 8)
        pltpu.async_copy(tok_hbm.at[idx_all.at[pl.ds(off1, _G1)]],
                         bufs[b].at[pl.ds(_G0, _G1)], gsems[b])

    def drain_gather(b):
        pltpu.make_async_copy(tok_hbm.at[pl.ds(0, _L)], bufs[b],
                              gsems[b]).wait()

    def drain_out(b):
        pltpu.make_async_copy(bufs[b], out_hbm.at[pl.ds(0, _L)],
                              osems[b]).wait()

    for b in range(_DEPTH):
        start_gather(b, b)

    def wave(w, carry):
        for b in range(_NBUF):
            c = w * _NBUF + b
            drain_gather(b)

            def add_row(l, c2):
                buf = bufs[b]
                for k in range(_D // _LANES):
                    sl = pl.ds(k * _LANES, _LANES)
                    buf[l, sl] = buf[l, sl] + pos_v[l, sl]
                return c2

            lax.fori_loop(0, _L, add_row, 0, unroll=2)

            off = pl.multiple_of(base + c * _L, _L)
            pltpu.async_copy(bufs[b], out_hbm.at[pl.ds(off, _L)], osems[b])

            b2_ = (b + _DEPTH) % _NBUF

            @pl.when(c + _DEPTH < seqs_per_w)
            def _():
                @pl.when(c >= _NBUF - _DEPTH)
                def _():
                    drain_out(b2_)
                start_gather(c + _DEPTH, b2_)
        return carry

    lax.fori_loop(0, seqs_per_w // _NBUF, wave, 0)
    for b in range(_NBUF):
        drain_out(b)


def kernel(x, token_table, pos_table):
    B, L = x.shape
    V, D = token_table.shape
    assert L == _L and D == _D
    info = plsc.get_sparse_core_info()
    nw = info.num_cores * info.num_subcores
    assert B % nw == 0
    seqs_per_w = B // nw

    flat_idx = x.reshape(B * L).astype(jnp.int32)
    mesh = plsc.VectorSubcoreMesh(core_axis_name="c", subcore_axis_name="s")

    sems = [pltpu.SemaphoreType.DMA] * (2 * _NBUF)
    emb = functools.partial(
        pl.kernel,
        mesh=mesh,
        out_type=jax.ShapeDtypeStruct((B * L, D), jnp.float32),
        scratch_types=[
            pltpu.VMEM((seqs_per_w * _L,), jnp.int32),
            pltpu.VMEM((_L, _D), jnp.float32),
        ] + [pltpu.VMEM((_L, _D), jnp.float32) for _ in range(_NBUF)]
          + sems,
        compiler_params=pltpu.CompilerParams(use_tc_tiling_on_sc=False),
    )(functools.partial(_emb_body, seqs_per_w=seqs_per_w,
                        num_cores=info.num_cores))

    out = emb(token_table, flat_idx, pos_table)
    return out.reshape(B, L, D)


# speedup vs baseline: 1.5403x; 1.3089x over previous
import functools

import jax
import jax.numpy as jnp
from jax import lax
from jax.experimental import pallas as pl
from jax.experimental.pallas import tpu as pltpu
from jax.experimental.pallas import tpu_sc as plsc

_L = 200
_D = 64
_LANES = 16
_NBUF = 4
_DEPTH = 2
_G0, _G1 = 128, _L - 128


def _emb_body(tok_hbm, idx_hbm, pos_hbm, out_hbm,
              idx_all, pos_v, b0, b1, b2, b3,
              g0, g1, g2, g3, o0, o1, o2, o3,
              *, seqs_per_w, num_cores):
    bufs = (b0, b1, b2, b3)
    gsems = (g0, g1, g2, g3)
    osems = (o0, o1, o2, o3)
    nrows = seqs_per_w * _L
    wid = lax.axis_index("s") * num_cores + lax.axis_index("c")
    base = pl.multiple_of(wid * nrows, _L)

    pltpu.sync_copy(pos_hbm, pos_v)
    pltpu.sync_copy(idx_hbm.at[pl.ds(base, nrows)], idx_all)

    def start_gather(c, b):
        off = pl.multiple_of(c * _L, 8)
        pltpu.async_copy(tok_hbm.at[idx_all.at[pl.ds(off, _G0)]],
                         bufs[b].at[pl.ds(0, _G0)], gsems[b])
        off1 = pl.multiple_of(c * _L + _G0, 8)
        pltpu.async_copy(tok_hbm.at[idx_all.at[pl.ds(off1, _G1)]],
                         bufs[b].at[pl.ds(_G0, _G1)], gsems[b])

    def drain_gather(b):
        pltpu.make_async_copy(tok_hbm.at[pl.ds(0, _L)], bufs[b],
                              gsems[b]).wait()

    def drain_out(b):
        pltpu.make_async_copy(bufs[b], out_hbm.at[pl.ds(0, _L), pl.ds(0, _D)],
                              osems[b]).wait()

    for b in range(_DEPTH):
        start_gather(b, b)

    def wave(w, carry):
        for b in range(_NBUF):
            c = w * _NBUF + b
            drain_gather(b)

            def add_row(l, c2):
                buf = bufs[b]
                for k in range(_D // _LANES):
                    sl = pl.ds(k * _LANES, _LANES)
                    buf[l, sl] = buf[l, sl] + pos_v[l, sl]
                return c2

            lax.fori_loop(0, _L, add_row, 0, unroll=2)

            off = pl.multiple_of(base + c * _L, _L)
            pltpu.async_copy(bufs[b],
                             out_hbm.at[pl.ds(off, _L), pl.ds(0, _D)],
                             osems[b])

            b2_ = (b + _DEPTH) % _NBUF

            @pl.when(c + _DEPTH < seqs_per_w)
            def _():
                @pl.when(c >= _NBUF - _DEPTH)
                def _():
                    drain_out(b2_)
                start_gather(c + _DEPTH, b2_)
        return carry

    lax.fori_loop(0, seqs_per_w // _NBUF, wave, 0)
    for b in range(_NBUF):
        drain_out(b)


def kernel(x, token_table, pos_table):
    B, L = x.shape
    V, D = token_table.shape
    assert L == _L and D == _D
    info = plsc.get_sparse_core_info()
    nw = info.num_cores * info.num_subcores
    assert B % nw == 0
    seqs_per_w = B // nw

    tok2 = jnp.pad(token_table, ((0, 0), (0, _D))).reshape(2 * V, D)
    flat_idx = x.reshape(B * L).astype(jnp.int32) * 2
    mesh = plsc.VectorSubcoreMesh(core_axis_name="c", subcore_axis_name="s")

    sems = [pltpu.SemaphoreType.DMA] * (2 * _NBUF)
    emb = functools.partial(
        pl.kernel,
        mesh=mesh,
        out_type=jax.ShapeDtypeStruct((B * L, 2 * D), jnp.float32),
        scratch_types=[
            pltpu.VMEM((seqs_per_w * _L,), jnp.int32),
            pltpu.VMEM((_L, _D), jnp.float32),
        ] + [pltpu.VMEM((_L, _D), jnp.float32) for _ in range(_NBUF)]
          + sems,
        compiler_params=pltpu.CompilerParams(use_tc_tiling_on_sc=False),
    )(functools.partial(_emb_body, seqs_per_w=seqs_per_w,
                        num_cores=info.num_cores))

    out = emb(tok2, flat_idx, pos_table)
    return out[:, :_D].reshape(B, L, D)


# speedup vs baseline: 1.8608x; 1.2081x over previous
import functools

import jax
import jax.numpy as jnp
from jax import lax
from jax.experimental import pallas as pl
from jax.experimental.pallas import tpu as pltpu
from jax.experimental.pallas import tpu_sc as plsc

_L = 200
_D = 64
_LANES = 16
_SEQ_PER_CHUNK = 2
_CL = _SEQ_PER_CHUNK * _L
_NBUF = 3
_DEPTH = 2


def _emb_body(tok_hbm, idx_hbm, pos_hbm, out_hbm,
              idx_all, pos_v, b0, b1, b2,
              g0, g1, g2, o0, o1, o2,
              *, seqs_per_w, num_cores):
    bufs = (b0, b1, b2)
    gsems = (g0, g1, g2)
    osems = (o0, o1, o2)
    nrows = seqs_per_w * _L
    nchunks = nrows // _CL
    wid = lax.axis_index("s") * num_cores + lax.axis_index("c")
    base = pl.multiple_of(wid * nrows, _CL)

    pltpu.sync_copy(pos_hbm, pos_v)
    pltpu.sync_copy(idx_hbm.at[pl.ds(base, nrows)], idx_all)

    def start_gather(c, b):
        off = pl.multiple_of(c * _CL, 8)
        pltpu.async_copy(tok_hbm.at[idx_all.at[pl.ds(off, _CL)]],
                         bufs[b], gsems[b])

    def drain_gather(b):
        pltpu.make_async_copy(tok_hbm.at[pl.ds(0, _CL)], bufs[b],
                              gsems[b]).wait()

    def start_out(c, b):
        off = pl.multiple_of(base + c * _CL, _CL)
        pltpu.async_copy(bufs[b],
                         out_hbm.at[pl.ds(off, _CL), pl.ds(0, _D)],
                         osems[b])

    def drain_out(b):
        pltpu.make_async_copy(bufs[b], out_hbm.at[pl.ds(0, _CL), pl.ds(0, _D)],
                              osems[b]).wait()

    def add_pos(b):
        def add_row(l, c2):
            buf = bufs[b]
            for k in range(_D // _LANES):
                sl = pl.ds(k * _LANES, _LANES)
                p = pos_v[l, sl]
                for s in range(_SEQ_PER_CHUNK):
                    r = s * _L + l
                    buf[r, sl] = buf[r, sl] + p
            return c2
        lax.fori_loop(0, _L, add_row, 0, unroll=2)

    def process(c, b):
        drain_gather(b)
        add_pos(b)
        start_out(c, b)
        b2_ = (b + _DEPTH) % _NBUF

        @pl.when(c + _DEPTH < nchunks)
        def _():
            @pl.when(c >= 1)
            def _():
                drain_out(b2_)
            start_gather(c + _DEPTH, b2_)

    for b in range(_DEPTH):
        start_gather(b, b)

    def wave(w, carry):
        for b in range(_NBUF):
            process(w * _NBUF + b, b)
        return carry

    lax.fori_loop(0, (nchunks - 1) // _NBUF, wave, 0)
    process(nchunks - 1, (nchunks - 1) % _NBUF)
    for b in range(_NBUF):
        drain_out(b)


def kernel(x, token_table, pos_table):
    B, L = x.shape
    V, D = token_table.shape
    assert L == _L and D == _D
    info = plsc.get_sparse_core_info()
    nw = info.num_cores * info.num_subcores
    assert B % nw == 0
    seqs_per_w = B // nw

    tok2 = jnp.pad(token_table, ((0, 0), (0, _D))).reshape(2 * V, D)
    flat_idx = x.reshape(B * L).astype(jnp.int32) * 2
    mesh = plsc.VectorSubcoreMesh(core_axis_name="c", subcore_axis_name="s")

    sems = [pltpu.SemaphoreType.DMA] * (2 * _NBUF)
    emb = functools.partial(
        pl.kernel,
        mesh=mesh,
        out_type=jax.ShapeDtypeStruct((B * L, 2 * D), jnp.float32),
        scratch_types=[
            pltpu.VMEM((seqs_per_w * _L,), jnp.int32),
            pltpu.VMEM((_L, _D), jnp.float32),
        ] + [pltpu.VMEM((_CL, _D), jnp.float32) for _ in range(_NBUF)]
          + sems,
        compiler_params=pltpu.CompilerParams(use_tc_tiling_on_sc=False),
    )(functools.partial(_emb_body, seqs_per_w=seqs_per_w,
                        num_cores=info.num_cores))

    out = emb(tok2, flat_idx, pos_table)
    return out[:, :_D].reshape(B, L, D)
